# baseline (speedup 1.0000x reference)
import functools

import jax
import jax.numpy as jnp
import numpy as np
from jax import lax
from jax.experimental import pallas as pl
from jax.experimental.pallas import tpu as pltpu
from jax.experimental.pallas import tpu_sc as plsc

_N = 100000
_D = 128
_B = 10000
_T = 32

_NC = 2
_NS = 16
_NW = _NC * _NS
_BPAD = 10240
_BPW = _BPAD // _NW
_IDXCHUNK = 128


def _leaky(x):
    return jnp.where(x >= 0, x, 0.01 * x)


def _hq_body(h_ref, qwt_ref, qb_ref, o_ref):
    x = jnp.dot(h_ref[...], qwt_ref[...], preferred_element_type=jnp.float32)
    x = _leaky(x + qb_ref[...])
    ua = lax.bitcast_convert_type(
        x[:, :_D // 2].astype(jnp.bfloat16), jnp.uint16).astype(jnp.uint32)
    ub = lax.bitcast_convert_type(
        x[:, _D // 2:].astype(jnp.bfloat16), jnp.uint16).astype(jnp.uint32)
    o_ref[...] = ua | (ub << 16)


def _hq_precompute(h, q_wt, q_b2):
    blk = 2000
    return pl.pallas_call(
        _hq_body,
        grid=(_N // blk,),
        in_specs=[
            pl.BlockSpec((blk, _D), lambda i: (i, 0)),
            pl.BlockSpec((_D, _D), lambda i: (0, 0)),
            pl.BlockSpec((1, _D), lambda i: (0, 0)),
        ],
        out_specs=pl.BlockSpec((blk, _D // 2), lambda i: (i, 0)),
        out_shape=jax.ShapeDtypeStruct((_N, _D // 2), jnp.uint32),
    )(h, q_wt, q_b2)


def _out_body(hn_ref, agg_ref, w_ref, w1_ref, w2_ref, b_ref, o_ref):
    wsum = jnp.sum(w_ref[...], axis=1, keepdims=True)
    agg = agg_ref[...] / jnp.where(wsum == 0.0, 1.0, wsum)
    x = jnp.dot(hn_ref[...], w1_ref[...], preferred_element_type=jnp.float32)
    x = x + jnp.dot(agg, w2_ref[...], preferred_element_type=jnp.float32)
    x = _leaky(x + b_ref[...])
    nrm = jnp.sqrt(jnp.sum(x * x, axis=1, keepdims=True))
    o_ref[...] = x / jnp.where(nrm == 0.0, 1.0, nrm)


def _out_layer(hn, agg, w_p, w1t, w2t, w_b2):
    blk = 2048
    return pl.pallas_call(
        _out_body,
        grid=(_BPAD // blk,),
        in_specs=[
            pl.BlockSpec((blk, _D), lambda i: (i, 0)),
            pl.BlockSpec((blk, _D), lambda i: (i, 0)),
            pl.BlockSpec((blk, _T), lambda i: (i, 0)),
            pl.BlockSpec((_D, _D), lambda i: (0, 0)),
            pl.BlockSpec((_D, _D), lambda i: (0, 0)),
            pl.BlockSpec((1, _D), lambda i: (0, 0)),
        ],
        out_specs=pl.BlockSpec((blk, _D), lambda i: (i, 0)),
        out_shape=jax.ShapeDtypeStruct((_BPAD, _D), jnp.float32),
    )(hn, agg, w_p, w1t, w2t, w_b2)


def _wid():
    return lax.axis_index("s") * _NC + lax.axis_index("c")


def _chunked_row_gather(table_hbm, idx_v, idx_lo, dst_v, nrows, sem):
    copies = []
    for lo in range(0, nrows, _IDXCHUNK):
        n = min(_IDXCHUNK, nrows - lo)
        copies.append(pltpu.async_copy(
            table_hbm.at[idx_v.at[pl.ds(idx_lo + lo, n)]],
            dst_v.at[pl.ds(lo, n)], sem))
    return copies


_GB = 4
_BIDX = _GB * _T

_CHPERM = np.zeros(_D, dtype=np.int32)
for _g in range(4):
    for _i in range(16):
        _CHPERM[16 * _g + _i] = 32 * _g + _i
        _CHPERM[64 + 16 * _g + _i] = 32 * _g + 16 + _i


def _sc_agg_body(hq_hbm, nb_hbm, w_hbm,
                 agg_out,
                 nb_v, w_v, agg_v, nbr_v,
                 sem_nb, sem_w, sem0, sem1):
    base = _wid() * _BPW
    half = _BPW // 2
    bh = half // _GB
    cp_nb = pltpu.async_copy(nb_hbm.at[pl.ds(base * _T, _BPW * _T)], nb_v,
                             sem_nb)
    cp_w = pltpu.async_copy(w_hbm.at[pl.ds(base * _T, _BPW * _T)], w_v, sem_w)
    cp_nb.wait()
    cp_w.wait()

    sems = (sem0, sem1)

    def issue(bidx, s):
        pltpu.async_copy(
            hq_hbm.at[nb_v.at[pl.ds(bidx * _BIDX, _BIDX)]],
            nbr_v.at[s], sems[s])

    def drain(s):
        pltpu.make_async_copy(
            hq_hbm.at[nb_v.at[pl.ds(0, _BIDX)]], nbr_v.at[s], sems[s]).wait()

    def compute_row(i, local_i, buf, r):
        wr0 = w_v[pl.ds(i * _T, 16)]
        wr1 = w_v[pl.ds(i * _T + 16, 16)]
        acc = [jnp.zeros((16,), jnp.float32) for _ in range(8)]
        dnums = lax.GatherDimensionNumbers(
            offset_dims=(), collapsed_slice_dims=(0,), start_index_map=(0,))
        for t in range(_T):
            src = wr0 if t < 16 else wr1
            wt = lax.gather(src, jnp.full((16, 1), t % 16, jnp.int32),
                            dnums, slice_sizes=(1,),
                            mode=lax.GatherScatterMode.PROMISE_IN_BOUNDS)
            for g in range(4):
                u = nbr_v[buf, r * _T + t, pl.ds(g * 16, 16)]
                fe = plsc.bitcast(u << 16, jnp.float32)
                fo = plsc.bitcast(u, jnp.float32)
                acc[2 * g] = acc[2 * g] + wt * fe
                acc[2 * g + 1] = acc[2 * g + 1] + wt * fo
        for c in range(8):
            agg_v[pl.ds(local_i * _D + c * 16, 16)] = acc[c]

    for hh in range(2):
        hlo = hh * half
        b0 = hh * bh
        issue(b0, 0)
        issue(b0 + 1, 1)

        def body(k, carry, b0=b0, hlo=hlo):
            for s in range(2):
                bidx = b0 + 2 * k + s
                drain(s)
                for r in range(_GB):
                    i = bidx * _GB + r
                    compute_row(i, i - hlo, s, r)

                @pl.when(bidx + 2 < b0 + bh)
                def _():
                    issue(bidx + 2, s)
            return carry

        lax.fori_loop(0, bh // 2, body, 0)

        pltpu.sync_copy(agg_v,
                        agg_out.at[pl.ds((base + hlo) * _D, half * _D)])


def _sc_aggregate(hq, nb_p, w_p_flat):
    mesh = plsc.VectorSubcoreMesh(core_axis_name="c", subcore_axis_name="s")
    fn = functools.partial(
        pl.kernel,
        out_type=jax.ShapeDtypeStruct((_BPAD * _D,), jnp.float32),
        mesh=mesh,
        scratch_types=[
            pltpu.VMEM((_BPW * _T,), jnp.int32),
            pltpu.VMEM((_BPW * _T,), jnp.float32),
            pltpu.VMEM((_BPW // 2 * _D,), jnp.float32),
            pltpu.VMEM((2, _BIDX, _D // 2), jnp.uint32),
            pltpu.SemaphoreType.DMA,
            pltpu.SemaphoreType.DMA,
            pltpu.SemaphoreType.DMA,
            pltpu.SemaphoreType.DMA,
        ],
        compiler_params=pltpu.CompilerParams(needs_layout_passes=False,
                                             use_tc_tiling_on_sc=False),
    )(_sc_agg_body)
    return fn(hq, nb_p, w_p_flat)


def _sc_perm_body(src_hbm, perm_hbm, out_hbm, idx_v, rows_v, sem):
    base = _wid() * _BPW
    pltpu.sync_copy(perm_hbm.at[pl.ds(base, _BPW)], idx_v)
    for cp in _chunked_row_gather(src_hbm, idx_v, 0, rows_v, _BPW, sem):
        cp.wait()
    pltpu.sync_copy(rows_v, out_hbm.at[pl.ds(base, _BPW)])


def _sc_perm_gather(h_new, perm_p):
    mesh = plsc.VectorSubcoreMesh(core_axis_name="c", subcore_axis_name="s")
    fn = functools.partial(
        pl.kernel,
        out_type=jax.ShapeDtypeStruct((_BPAD, _D), jnp.float32),
        mesh=mesh,
        scratch_types=[
            pltpu.VMEM((_BPW,), jnp.int32),
            pltpu.VMEM((_BPW, _D), jnp.float32),
            pltpu.SemaphoreType.DMA,
        ],
    )(_sc_perm_body)
    return fn(h_new, perm_p)


def kernel(h, nodeset, nb_nodes, nb_weights, Q_w, Q_b, W_w, W_b):
    b, t = nb_nodes.shape
    pad = _BPAD - b
    win = jnp.zeros((_N,), jnp.int32).at[nodeset].max(
        jnp.arange(b, dtype=jnp.int32))
    perm = win[nodeset]
    spread = jnp.arange(pad, dtype=jnp.int32)
    perm_p = jnp.concatenate([perm, spread])
    node_p = jnp.concatenate([nodeset, spread])
    nb_p = jnp.concatenate(
        [nb_nodes.reshape(-1),
         jnp.arange(pad * t, dtype=jnp.int32) % _N])
    w_p_flat = jnp.concatenate(
        [nb_weights.reshape(-1), jnp.ones((pad * t,), jnp.float32)])

    hq = _hq_precompute(h, Q_w.T[:, _CHPERM], Q_b[_CHPERM].reshape(1, _D))
    agg_flat = _sc_aggregate(hq, nb_p, w_p_flat)
    hn = _sc_perm_gather(h, node_p)
    agg = agg_flat.reshape(_BPAD, _D)
    h_new = _out_layer(hn, agg, w_p_flat.reshape(_BPAD, _T),
                       W_w[:, :_D].T, W_w[:, _D:].T, W_b.reshape(1, _D))
    out = _sc_perm_gather(h_new, perm_p)
    return out[:b]


# speedup vs baseline: 1.2847x; 1.2847x over previous
import functools

import jax
import jax.numpy as jnp
import numpy as np
from jax import lax
from jax.experimental import pallas as pl
from jax.experimental.pallas import tpu as pltpu
from jax.experimental.pallas import tpu_sc as plsc

_N = 100000
_D = 128
_B = 10000
_T = 32

_NC = 2
_NS = 16
_NW = _NC * _NS
_BPAD = 10240
_BPW = _BPAD // _NW
_IDXCHUNK = 128


def _leaky(x):
    return jnp.where(x >= 0, x, 0.01 * x)


def _pack_u32(x):
    lo = lax.bitcast_convert_type(
        x[:, :_D // 2].astype(jnp.bfloat16), jnp.uint16).astype(jnp.uint32)
    hi = lax.bitcast_convert_type(
        x[:, _D // 2:].astype(jnp.bfloat16), jnp.uint16).astype(jnp.uint32)
    return lo | (hi << 16)


def _hq_body(ha_ref, hb_ref, qwt_ref, qb_ref, o_ref):
    xa = _leaky(jnp.dot(ha_ref[...], qwt_ref[...],
                        preferred_element_type=jnp.float32) + qb_ref[...])
    xb = _leaky(jnp.dot(hb_ref[...], qwt_ref[...],
                        preferred_element_type=jnp.float32) + qb_ref[...])
    o_ref[...] = jnp.concatenate([_pack_u32(xa), _pack_u32(xb)], axis=1)


def _hq_precompute(h, q_wt, q_b2):
    blk = 2000
    nb2 = _N // 2 // blk
    return pl.pallas_call(
        _hq_body,
        grid=(nb2,),
        in_specs=[
            pl.BlockSpec((blk, _D), lambda i: (i, 0)),
            pl.BlockSpec((blk, _D), lambda i: (i + nb2, 0)),
            pl.BlockSpec((_D, _D), lambda i: (0, 0)),
            pl.BlockSpec((1, _D), lambda i: (0, 0)),
        ],
        out_specs=pl.BlockSpec((blk, _D), lambda i: (i, 0)),
        out_shape=jax.ShapeDtypeStruct((_N // 2, _D), jnp.uint32),
    )(h, h, q_wt, q_b2)


def _out_body(hn_ref, agg_ref, w_ref, w1_ref, w2_ref, b_ref, o_ref):
    wsum = jnp.sum(w_ref[...], axis=1, keepdims=True)
    agg = agg_ref[...] / jnp.where(wsum == 0.0, 1.0, wsum)
    x = jnp.dot(hn_ref[...], w1_ref[...], preferred_element_type=jnp.float32)
    x = x + jnp.dot(agg, w2_ref[...], preferred_element_type=jnp.float32)
    x = _leaky(x + b_ref[...])
    nrm = jnp.sqrt(jnp.sum(x * x, axis=1, keepdims=True))
    o_ref[...] = x / jnp.where(nrm == 0.0, 1.0, nrm)


def _out_layer(hn, agg, w_p, w1t, w2t, w_b2):
    blk = 2048
    return pl.pallas_call(
        _out_body,
        grid=(_BPAD // blk,),
        in_specs=[
            pl.BlockSpec((blk, _D), lambda i: (i, 0)),
            pl.BlockSpec((blk, _D), lambda i: (i, 0)),
            pl.BlockSpec((blk, _T), lambda i: (i, 0)),
            pl.BlockSpec((_D, _D), lambda i: (0, 0)),
            pl.BlockSpec((_D, _D), lambda i: (0, 0)),
            pl.BlockSpec((1, _D), lambda i: (0, 0)),
        ],
        out_specs=pl.BlockSpec((blk, _D), lambda i: (i, 0)),
        out_shape=jax.ShapeDtypeStruct((_BPAD, _D), jnp.float32),
    )(hn, agg, w_p, w1t, w2t, w_b2)


def _wid():
    return lax.axis_index("s") * _NC + lax.axis_index("c")


def _chunked_row_gather(table_hbm, idx_v, idx_lo, dst_v, nrows, sem):
    copies = []
    for lo in range(0, nrows, _IDXCHUNK):
        n = min(_IDXCHUNK, nrows - lo)
        copies.append(pltpu.async_copy(
            table_hbm.at[idx_v.at[pl.ds(idx_lo + lo, n)]],
            dst_v.at[pl.ds(lo, n)], sem))
    return copies


_GB = 4
_BIDX = _GB * _T

_CHPERM = np.zeros(_D, dtype=np.int32)
for _g in range(4):
    for _i in range(16):
        _CHPERM[16 * _g + _i] = 32 * _g + _i
        _CHPERM[64 + 16 * _g + _i] = 32 * _g + 16 + _i


def _sc_agg_body(hq_hbm, nb_hbm, w_hbm,
                 agg_out,
                 nb_v, w_v, agg_v, nbr_v,
                 sem_nb, sem_w, sem0, sem1):
    base = _wid() * _BPW
    half = _BPW // 2
    bh = half // _GB
    cp_nb = pltpu.async_copy(nb_hbm.at[pl.ds(base * _T, _BPW * _T)], nb_v,
                             sem_nb)
    cp_w = pltpu.async_copy(w_hbm.at[pl.ds(base * _T, _BPW * _T)], w_v, sem_w)
    cp_nb.wait()
    cp_w.wait()

    sems = (sem0, sem1)

    def issue(bidx, s):
        pltpu.async_copy(
            hq_hbm.at[nb_v.at[pl.ds(bidx * _BIDX, _BIDX)]],
            nbr_v.at[s], sems[s])

    def drain(s):
        pltpu.make_async_copy(
            hq_hbm.at[nb_v.at[pl.ds(0, _BIDX)]], nbr_v.at[s], sems[s]).wait()

    def compute_row(i, local_i, buf, r):
        wr0 = w_v[pl.ds(i * _T, 16)]
        wr1 = w_v[pl.ds(i * _T + 16, 16)]
        acc = [jnp.zeros((16,), jnp.float32) for _ in range(8)]
        dnums = lax.GatherDimensionNumbers(
            offset_dims=(), collapsed_slice_dims=(0,), start_index_map=(0,))
        for t in range(_T):
            src = wr0 if t < 16 else wr1
            wt = lax.gather(src, jnp.full((16, 1), t % 16, jnp.int32),
                            dnums, slice_sizes=(1,),
                            mode=lax.GatherScatterMode.PROMISE_IN_BOUNDS)
            for g in range(4):
                u = nbr_v[buf, r * _T + t, pl.ds(g * 16, 16)]
                fe = plsc.bitcast(u << 16, jnp.float32)
                fo = plsc.bitcast(u, jnp.float32)
                acc[2 * g] = acc[2 * g] + wt * fe
                acc[2 * g + 1] = acc[2 * g + 1] + wt * fo
        for c in range(8):
            agg_v[pl.ds(local_i * _D + c * 16, 16)] = acc[c]

    for hh in range(2):
        hlo = hh * half
        b0 = hh * bh
        issue(b0, 0)
        issue(b0 + 1, 1)

        def body(k, carry, b0=b0, hlo=hlo):
            for s in range(2):
                bidx = b0 + 2 * k + s
                drain(s)
                for r in range(_GB):
                    i = bidx * _GB + r
                    compute_row(i, i - hlo, s, r)

                @pl.when(bidx + 2 < b0 + bh)
                def _():
                    issue(bidx + 2, s)
            return carry

        lax.fori_loop(0, bh // 2, body, 0)

        pltpu.sync_copy(agg_v,
                        agg_out.at[pl.ds((base + hlo) * _D, half * _D)])


def _sc_aggregate(hq, nb_p, w_p_flat):
    mesh = plsc.VectorSubcoreMesh(core_axis_name="c", subcore_axis_name="s")
    fn = functools.partial(
        pl.kernel,
        out_type=jax.ShapeDtypeStruct((_BPAD * _D,), jnp.float32),
        mesh=mesh,
        scratch_types=[
            pltpu.VMEM((_BPW * _T,), jnp.int32),
            pltpu.VMEM((_BPW * _T,), jnp.float32),
            pltpu.VMEM((_BPW // 2 * _D,), jnp.float32),
            pltpu.VMEM((2, _BIDX, _D // 2), jnp.uint32),
            pltpu.SemaphoreType.DMA,
            pltpu.SemaphoreType.DMA,
            pltpu.SemaphoreType.DMA,
            pltpu.SemaphoreType.DMA,
        ],
        compiler_params=pltpu.CompilerParams(needs_layout_passes=False,
                                             use_tc_tiling_on_sc=False),
    )(_sc_agg_body)
    return fn(hq, nb_p, w_p_flat)


def _sc_perm_body(src_hbm, perm_hbm, out_hbm, idx_v, rows_v, sem):
    base = _wid() * _BPW
    pltpu.sync_copy(perm_hbm.at[pl.ds(base, _BPW)], idx_v)
    for cp in _chunked_row_gather(src_hbm, idx_v, 0, rows_v, _BPW, sem):
        cp.wait()
    pltpu.sync_copy(rows_v, out_hbm.at[pl.ds(base, _BPW)])


def _sc_perm_gather(h_new, perm_p):
    mesh = plsc.VectorSubcoreMesh(core_axis_name="c", subcore_axis_name="s")
    fn = functools.partial(
        pl.kernel,
        out_type=jax.ShapeDtypeStruct((_BPAD, _D), jnp.float32),
        mesh=mesh,
        scratch_types=[
            pltpu.VMEM((_BPW,), jnp.int32),
            pltpu.VMEM((_BPW, _D), jnp.float32),
            pltpu.SemaphoreType.DMA,
        ],
    )(_sc_perm_body)
    return fn(h_new, perm_p)


def kernel(h, nodeset, nb_nodes, nb_weights, Q_w, Q_b, W_w, W_b):
    b, t = nb_nodes.shape
    pad = _BPAD - b
    win = jnp.zeros((_N,), jnp.int32).at[nodeset].max(
        jnp.arange(b, dtype=jnp.int32))
    perm = win[nodeset]
    spread = jnp.arange(pad, dtype=jnp.int32)
    perm_p = jnp.concatenate([perm, spread])
    node_p = jnp.concatenate([nodeset, spread])
    nb_p = jnp.concatenate(
        [nb_nodes.reshape(-1),
         jnp.arange(pad * t, dtype=jnp.int32) % _N])
    nb_p = jnp.where(nb_p < _N // 2, 2 * nb_p, 2 * nb_p - _N + 1)
    w_p_flat = jnp.concatenate(
        [nb_weights.reshape(-1), jnp.ones((pad * t,), jnp.float32)])

    hq = _hq_precompute(h, Q_w.T[:, _CHPERM], Q_b[_CHPERM].reshape(1, _D))
    agg_flat = _sc_aggregate(hq.reshape(_N, _D // 2), nb_p, w_p_flat)
    hn = _sc_perm_gather(h, node_p)
    agg = agg_flat.reshape(_BPAD, _D)
    h_new = _out_layer(hn, agg, w_p_flat.reshape(_BPAD, _T),
                       W_w[:, :_D].T, W_w[:, _D:].T, W_b.reshape(1, _D))
    out = _sc_perm_gather(h_new, perm_p)
    return out[:b]


# speedup vs baseline: 1.3084x; 1.0185x over previous
import functools

import jax
import jax.numpy as jnp
import numpy as np
from jax import lax
from jax.experimental import pallas as pl
from jax.experimental.pallas import tpu as pltpu
from jax.experimental.pallas import tpu_sc as plsc

_N = 100000
_D = 128
_B = 10000
_T = 32

_NC = 2
_NS = 16
_NW = _NC * _NS
_BPAD = 10240
_BPW = _BPAD // _NW
_IDXCHUNK = 128


def _leaky(x):
    return jnp.where(x >= 0, x, 0.01 * x)


def _pack_u32(x):
    lo = lax.bitcast_convert_type(
        x[:, :_D // 2].astype(jnp.bfloat16), jnp.uint16).astype(jnp.uint32)
    hi = lax.bitcast_convert_type(
        x[:, _D // 2:].astype(jnp.bfloat16), jnp.uint16).astype(jnp.uint32)
    return lo | (hi << 16)


def _hq_body(ha_ref, hb_ref, qwt_ref, qb_ref, o_ref):
    xa = _leaky(jnp.dot(ha_ref[...], qwt_ref[...],
                        preferred_element_type=jnp.float32) + qb_ref[...])
    xb = _leaky(jnp.dot(hb_ref[...], qwt_ref[...],
                        preferred_element_type=jnp.float32) + qb_ref[...])
    o_ref[...] = jnp.concatenate([_pack_u32(xa), _pack_u32(xb)], axis=1)


def _hq_precompute(h, q_wt, q_b2):
    blk = 2000
    nb2 = _N // 2 // blk
    return pl.pallas_call(
        _hq_body,
        grid=(nb2,),
        in_specs=[
            pl.BlockSpec((blk, _D), lambda i: (i, 0)),
            pl.BlockSpec((blk, _D), lambda i: (i + nb2, 0)),
            pl.BlockSpec((_D, _D), lambda i: (0, 0)),
            pl.BlockSpec((1, _D), lambda i: (0, 0)),
        ],
        out_specs=pl.BlockSpec((blk, _D), lambda i: (i, 0)),
        out_shape=jax.ShapeDtypeStruct((_N // 2, _D), jnp.uint32),
    )(h, h, q_wt, q_b2)


def _out_body(hn_ref, agg_ref, w_ref, w1_ref, w2_ref, b_ref, o_ref):
    wsum = jnp.sum(w_ref[...], axis=1, keepdims=True)
    agg = agg_ref[...] / jnp.where(wsum == 0.0, 1.0, wsum)
    x = jnp.dot(hn_ref[...], w1_ref[...], preferred_element_type=jnp.float32)
    x = x + jnp.dot(agg, w2_ref[...], preferred_element_type=jnp.float32)
    x = _leaky(x + b_ref[...])
    nrm = jnp.sqrt(jnp.sum(x * x, axis=1, keepdims=True))
    o_ref[...] = x / jnp.where(nrm == 0.0, 1.0, nrm)


def _out_layer(hn, agg, w_p, w1t, w2t, w_b2):
    blk = 2048
    return pl.pallas_call(
        _out_body,
        grid=(_BPAD // blk,),
        in_specs=[
            pl.BlockSpec((blk, _D), lambda i: (i, 0)),
            pl.BlockSpec((blk, _D), lambda i: (i, 0)),
            pl.BlockSpec((blk, _T), lambda i: (i, 0)),
            pl.BlockSpec((_D, _D), lambda i: (0, 0)),
            pl.BlockSpec((_D, _D), lambda i: (0, 0)),
            pl.BlockSpec((1, _D), lambda i: (0, 0)),
        ],
        out_specs=pl.BlockSpec((blk, _D), lambda i: (i, 0)),
        out_shape=jax.ShapeDtypeStruct((_BPAD, _D), jnp.float32),
    )(hn, agg, w_p, w1t, w2t, w_b2)


def _wid():
    return lax.axis_index("s") * _NC + lax.axis_index("c")


def _chunked_row_gather(table_hbm, idx_v, idx_lo, dst_v, nrows, sem):
    copies = []
    for lo in range(0, nrows, _IDXCHUNK):
        n = min(_IDXCHUNK, nrows - lo)
        copies.append(pltpu.async_copy(
            table_hbm.at[idx_v.at[pl.ds(idx_lo + lo, n)]],
            dst_v.at[pl.ds(lo, n)], sem))
    return copies


_GB = 4
_BIDX = _GB * _T

_CHPERM = np.zeros(_D, dtype=np.int32)
for _g in range(4):
    for _i in range(16):
        _CHPERM[16 * _g + _i] = 32 * _g + _i
        _CHPERM[64 + 16 * _g + _i] = 32 * _g + 16 + _i


_NRANGE = _N // _NS
_TBL = 6256
_HBP = _BPAD // 2


def _sc_agg_body(hq_hbm, nb_hbm, w_hbm, node_hbm,
                 agg_out, perm_out,
                 nb_v, w_v, agg_v, nbr_v,
                 table_v, scan_v, contrib_v, fin_v, fin_acc, shared_v,
                 sem_nb, sem_w, sem0, sem1):
    base = _wid() * _BPW
    half = _BPW // 2
    bh = half // _GB
    cp_nb = pltpu.async_copy(nb_hbm.at[pl.ds(base * _T, _BPW * _T)], nb_v,
                             sem_nb)
    cp_w = pltpu.async_copy(w_hbm.at[pl.ds(base * _T, _BPW * _T)], w_v, sem_w)
    cp_nb.wait()
    cp_w.wait()

    sems = (sem0, sem1)

    def issue(bidx, s):
        pltpu.async_copy(
            hq_hbm.at[nb_v.at[pl.ds(bidx * _BIDX, _BIDX)]],
            nbr_v.at[s], sems[s])

    def drain(s):
        pltpu.make_async_copy(
            hq_hbm.at[nb_v.at[pl.ds(0, _BIDX)]], nbr_v.at[s], sems[s]).wait()

    def compute_row(i, local_i, buf, r):
        wr0 = w_v[pl.ds(i * _T, 16)]
        wr1 = w_v[pl.ds(i * _T + 16, 16)]
        acc = [jnp.zeros((16,), jnp.float32) for _ in range(8)]
        dnums = lax.GatherDimensionNumbers(
            offset_dims=(), collapsed_slice_dims=(0,), start_index_map=(0,))
        for t in range(_T):
            src = wr0 if t < 16 else wr1
            wt = lax.gather(src, jnp.full((16, 1), t % 16, jnp.int32),
                            dnums, slice_sizes=(1,),
                            mode=lax.GatherScatterMode.PROMISE_IN_BOUNDS)
            for g in range(4):
                u = nbr_v[buf, r * _T + t, pl.ds(g * 16, 16)]
                fe = plsc.bitcast(u << 16, jnp.float32)
                fo = plsc.bitcast(u, jnp.float32)
                acc[2 * g] = acc[2 * g] + wt * fe
                acc[2 * g + 1] = acc[2 * g + 1] + wt * fo
        for c in range(8):
            agg_v[pl.ds(local_i * _D + c * 16, 16)] = acc[c]

    for hh in range(2):
        hlo = hh * half
        b0 = hh * bh
        issue(b0, 0)
        issue(b0 + 1, 1)

        def body(k, carry, b0=b0, hlo=hlo):
            for s in range(2):
                bidx = b0 + 2 * k + s
                drain(s)
                for r in range(_GB):
                    i = bidx * _GB + r
                    compute_row(i, i - hlo, s, r)

                @pl.when(bidx + 2 < b0 + bh)
                def _():
                    issue(bidx + 2, s)
            return carry

        lax.fori_loop(0, bh // 2, body, 0)

        pltpu.sync_copy(agg_v,
                        agg_out.at[pl.ds((base + hlo) * _D, half * _D)])

    sid = lax.axis_index("s")
    cid = lax.axis_index("c")
    lo = sid * _NRANGE
    iota16 = lax.iota(jnp.int32, 16)
    last15 = iota16 == 15
    zero16 = jnp.zeros((16,), jnp.int32)
    nxt_idx = jnp.minimum(iota16 + 1, 15).reshape(16, 1)
    dnums = lax.GatherDimensionNumbers(
        offset_dims=(), collapsed_slice_dims=(0,), start_index_map=(0,))

    def ztab(k, carry):
        table_v[pl.ds(k * 16, 16)] = zero16
        return carry

    lax.fori_loop(0, _TBL // 16, ztab, 0)

    def scan_a(ch, carry):
        pltpu.sync_copy(node_hbm.at[pl.ds(ch * 2000, 2000)],
                        scan_v.at[pl.ds(0, 2000)])

        def grp_a(g, carry2, ch=ch):
            idx = scan_v[pl.ds(g * 16, 16)]
            key = idx * 16384 + (ch * 2000 + g * 16 + iota16)
            ks = lax.sort(key)
            idn = lax.shift_right_logical(ks, 14)
            jv = ks & 16383
            nxt = lax.gather(idn, nxt_idx, dnums, slice_sizes=(1,),
                             mode=lax.GatherScatterMode.PROMISE_IN_BOUNDS)
            m = ((idn != nxt) | last15) & (idn >= lo) & (idn < lo + _NRANGE)
            loc = jnp.clip(idn - lo, 0, _NRANGE - 1)
            plsc.store_scatter(table_v, [loc], jv, mask=m)
            return carry2

        lax.fori_loop(0, 125, grp_a, 0)
        return carry

    lax.fori_loop(0, 5, scan_a, 0)

    def scan_b(ch, carry):
        pltpu.sync_copy(node_hbm.at[pl.ds(ch * 2048, 2048)], scan_v)

        def grp_b(g, carry2, ch=ch):
            idx = scan_v[pl.ds(g * 16, 16)]
            loc = jnp.clip(idx - lo, 0, _NRANGE - 1)
            val = plsc.load_gather(table_v, [loc])
            inr = (idx >= lo) & (idx < lo + _NRANGE)
            contrib_v[pl.ds(ch * 2048 + g * 16, 16)] = \
                jnp.where(inr, val + 1, 0)
            return carry2

        lax.fori_loop(0, 2048 // 16, grp_b, 0)
        return carry

    lax.fori_loop(0, 5, scan_b, 0)
    pltpu.sync_copy(contrib_v, shared_v.at[sid])
    plsc.subcore_barrier()

    fpw = _HBP // _NS
    off = cid * _HBP + sid * fpw
    for s in range(_NS):
        pltpu.sync_copy(shared_v.at[s, pl.ds(off, fpw)], fin_acc.at[s])

    def fing(g, carry):
        tot = fin_acc[0, pl.ds(g * 16, 16)]
        for s in range(1, _NS):
            tot = tot + fin_acc[s, pl.ds(g * 16, 16)]
        fin_v[pl.ds(g * 16, 16)] = tot - 1
        return carry

    lax.fori_loop(0, fpw // 16, fing, 0)
    pltpu.sync_copy(fin_v, perm_out.at[pl.ds(off, fpw)])


def _sc_aggregate(hq, nb_p, w_p_flat, node_p):
    mesh = plsc.VectorSubcoreMesh(core_axis_name="c", subcore_axis_name="s")
    fn = functools.partial(
        pl.kernel,
        out_type=(
            jax.ShapeDtypeStruct((_BPAD * _D,), jnp.float32),
            jax.ShapeDtypeStruct((_BPAD,), jnp.int32),
        ),
        mesh=mesh,
        scratch_types=[
            pltpu.VMEM((_BPW * _T,), jnp.int32),
            pltpu.VMEM((_BPW * _T,), jnp.float32),
            pltpu.VMEM((_BPW // 2 * _D,), jnp.float32),
            pltpu.VMEM((2, _BIDX, _D // 2), jnp.uint32),
            pltpu.VMEM((_TBL,), jnp.int32),
            pltpu.VMEM((2048,), jnp.int32),
            pltpu.VMEM((_BPAD,), jnp.int32),
            pltpu.VMEM((_HBP // _NS,), jnp.int32),
            pltpu.VMEM((_NS, _HBP // _NS), jnp.int32),
            pltpu.VMEM_SHARED((_NS, _BPAD), jnp.int32),
            pltpu.SemaphoreType.DMA,
            pltpu.SemaphoreType.DMA,
            pltpu.SemaphoreType.DMA,
            pltpu.SemaphoreType.DMA,
        ],
        compiler_params=pltpu.CompilerParams(needs_layout_passes=False,
                                             use_tc_tiling_on_sc=False),
    )(_sc_agg_body)
    return fn(hq, nb_p, w_p_flat, node_p)


def _sc_perm_body(src_hbm, perm_hbm, out_hbm, idx_v, rows_v, sem):
    base = _wid() * _BPW
    pltpu.sync_copy(perm_hbm.at[pl.ds(base, _BPW)], idx_v)
    for cp in _chunked_row_gather(src_hbm, idx_v, 0, rows_v, _BPW, sem):
        cp.wait()
    pltpu.sync_copy(rows_v, out_hbm.at[pl.ds(base, _BPW)])


def _sc_perm_gather(h_new, perm_p):
    mesh = plsc.VectorSubcoreMesh(core_axis_name="c", subcore_axis_name="s")
    fn = functools.partial(
        pl.kernel,
        out_type=jax.ShapeDtypeStruct((_BPAD, _D), jnp.float32),
        mesh=mesh,
        scratch_types=[
            pltpu.VMEM((_BPW,), jnp.int32),
            pltpu.VMEM((_BPW, _D), jnp.float32),
            pltpu.SemaphoreType.DMA,
        ],
    )(_sc_perm_body)
    return fn(h_new, perm_p)


def kernel(h, nodeset, nb_nodes, nb_weights, Q_w, Q_b, W_w, W_b):
    b, t = nb_nodes.shape
    pad = _BPAD - b
    spread = jnp.arange(pad, dtype=jnp.int32)
    node_p = jnp.concatenate([nodeset, spread])
    nb_p = jnp.concatenate(
        [nb_nodes.reshape(-1),
         jnp.arange(pad * t, dtype=jnp.int32) % _N])
    nb_p = jnp.where(nb_p < _N // 2, 2 * nb_p, 2 * nb_p - _N + 1)
    w_p_flat = jnp.concatenate(
        [nb_weights.reshape(-1), jnp.ones((pad * t,), jnp.float32)])

    hq = _hq_precompute(h, Q_w.T[:, _CHPERM], Q_b[_CHPERM].reshape(1, _D))
    agg_flat, perm_p = _sc_aggregate(hq.reshape(_N, _D // 2), nb_p,
                                     w_p_flat, node_p)
    hn = _sc_perm_gather(h, node_p)
    agg = agg_flat.reshape(_BPAD, _D)
    h_new = _out_layer(hn, agg, w_p_flat.reshape(_BPAD, _T),
                       W_w[:, :_D].T, W_w[:, _D:].T, W_b.reshape(1, _D))
    out = _sc_perm_gather(h_new, perm_p)
    return out[:b]


# speedup vs baseline: 1.3363x; 1.0214x over previous
import functools

import jax
import jax.numpy as jnp
import numpy as np
from jax import lax
from jax.experimental import pallas as pl
from jax.experimental.pallas import tpu as pltpu
from jax.experimental.pallas import tpu_sc as plsc

_N = 100000
_D = 128
_B = 10000
_T = 32

_NC = 2
_NS = 16
_NW = _NC * _NS
_BPAD = 10240
_BPW = _BPAD // _NW
_IDXCHUNK = 128


def _leaky(x):
    return jnp.where(x >= 0, x, 0.01 * x)


def _pack_u32(x):
    lo = lax.bitcast_convert_type(
        x[:, :_D // 2].astype(jnp.bfloat16), jnp.uint16).astype(jnp.uint32)
    hi = lax.bitcast_convert_type(
        x[:, _D // 2:].astype(jnp.bfloat16), jnp.uint16).astype(jnp.uint32)
    return lo | (hi << 16)


def _hq_body(ha_ref, hb_ref, qwt_ref, qb_ref, o_ref):
    xa = _leaky(jnp.dot(ha_ref[...], qwt_ref[...],
                        preferred_element_type=jnp.float32) + qb_ref[...])
    xb = _leaky(jnp.dot(hb_ref[...], qwt_ref[...],
                        preferred_element_type=jnp.float32) + qb_ref[...])
    o_ref[...] = jnp.concatenate([_pack_u32(xa), _pack_u32(xb)], axis=1)


def _hq_precompute(h, q_wt, q_b2):
    blk = 2000
    nb2 = _N // 2 // blk
    return pl.pallas_call(
        _hq_body,
        grid=(nb2,),
        in_specs=[
            pl.BlockSpec((blk, _D), lambda i: (i, 0)),
            pl.BlockSpec((blk, _D), lambda i: (i + nb2, 0)),
            pl.BlockSpec((_D, _D), lambda i: (0, 0)),
            pl.BlockSpec((1, _D), lambda i: (0, 0)),
        ],
        out_specs=pl.BlockSpec((blk, _D), lambda i: (i, 0)),
        out_shape=jax.ShapeDtypeStruct((_N // 2, _D), jnp.uint32),
    )(h, h, q_wt, q_b2)


def _out_body(hn_ref, agg_ref, w_ref, w1_ref, w2_ref, b_ref, o_ref):
    wsum = jnp.sum(w_ref[...], axis=1, keepdims=True)
    agg = agg_ref[...] / jnp.where(wsum == 0.0, 1.0, wsum)
    x = jnp.dot(hn_ref[...], w1_ref[...], preferred_element_type=jnp.float32)
    x = x + jnp.dot(agg, w2_ref[...], preferred_element_type=jnp.float32)
    x = _leaky(x + b_ref[...])
    nrm = jnp.sqrt(jnp.sum(x * x, axis=1, keepdims=True))
    o_ref[...] = x / jnp.where(nrm == 0.0, 1.0, nrm)


def _out_layer(hn, agg, w_p, w1t, w2t, w_b2):
    blk = 2048
    return pl.pallas_call(
        _out_body,
        grid=(_BPAD // blk,),
        in_specs=[
            pl.BlockSpec((blk, _D), lambda i: (i, 0)),
            pl.BlockSpec((blk, _D), lambda i: (i, 0)),
            pl.BlockSpec((blk, _T), lambda i: (i, 0)),
            pl.BlockSpec((_D, _D), lambda i: (0, 0)),
            pl.BlockSpec((_D, _D), lambda i: (0, 0)),
            pl.BlockSpec((1, _D), lambda i: (0, 0)),
        ],
        out_specs=pl.BlockSpec((blk, _D), lambda i: (i, 0)),
        out_shape=jax.ShapeDtypeStruct((_BPAD, _D), jnp.float32),
    )(hn, agg, w_p, w1t, w2t, w_b2)


def _wid():
    return lax.axis_index("s") * _NC + lax.axis_index("c")


def _chunked_row_gather(table_hbm, idx_v, idx_lo, dst_v, nrows, sem):
    copies = []
    for lo in range(0, nrows, _IDXCHUNK):
        n = min(_IDXCHUNK, nrows - lo)
        copies.append(pltpu.async_copy(
            table_hbm.at[idx_v.at[pl.ds(idx_lo + lo, n)]],
            dst_v.at[pl.ds(lo, n)], sem))
    return copies


_GB = 4
_BIDX = _GB * _T

_CHPERM = np.zeros(_D, dtype=np.int32)
for _g in range(4):
    for _i in range(16):
        _CHPERM[16 * _g + _i] = 32 * _g + _i
        _CHPERM[64 + 16 * _g + _i] = 32 * _g + 16 + _i


_NRANGE = _N // _NS
_TBL = 6256
_HBP = _BPAD // 2


def _sc_agg_body(hq_hbm, nb_hbm, w_hbm, node_hbm, h_hbm,
                 agg_out, perm_out, hn_out,
                 nb_v, w_v, agg_v, nbr_v, node_v, hn_v,
                 table_v, scan_v, contrib_v, fin_v, fin_acc, shared_v,
                 sem_nb, sem_w, sem_h, sem0, sem1):
    base = _wid() * _BPW
    half = _BPW // 2
    bh = half // _GB
    pltpu.sync_copy(node_hbm.at[pl.ds(base, _BPW)], node_v)
    cp_nb = pltpu.async_copy(nb_hbm.at[pl.ds(base * _T, _BPW * _T)], nb_v,
                             sem_nb)
    cp_w = pltpu.async_copy(w_hbm.at[pl.ds(base * _T, _BPW * _T)], w_v, sem_w)
    cp_nb.wait()
    cp_w.wait()

    sems = (sem0, sem1)

    def issue(bidx, s):
        pltpu.async_copy(
            hq_hbm.at[nb_v.at[pl.ds(bidx * _BIDX, _BIDX)]],
            nbr_v.at[s], sems[s])

    def drain(s):
        pltpu.make_async_copy(
            hq_hbm.at[nb_v.at[pl.ds(0, _BIDX)]], nbr_v.at[s], sems[s]).wait()

    def compute_row(i, local_i, buf, r):
        wr0 = w_v[pl.ds(i * _T, 16)]
        wr1 = w_v[pl.ds(i * _T + 16, 16)]
        acc = [jnp.zeros((16,), jnp.float32) for _ in range(8)]
        dnums = lax.GatherDimensionNumbers(
            offset_dims=(), collapsed_slice_dims=(0,), start_index_map=(0,))
        for t in range(_T):
            src = wr0 if t < 16 else wr1
            wt = lax.gather(src, jnp.full((16, 1), t % 16, jnp.int32),
                            dnums, slice_sizes=(1,),
                            mode=lax.GatherScatterMode.PROMISE_IN_BOUNDS)
            for g in range(4):
                u = nbr_v[buf, r * _T + t, pl.ds(g * 16, 16)]
                fe = plsc.bitcast(u << 16, jnp.float32)
                fo = plsc.bitcast(u, jnp.float32)
                acc[2 * g] = acc[2 * g] + wt * fe
                acc[2 * g + 1] = acc[2 * g + 1] + wt * fo
        for c in range(8):
            agg_v[pl.ds(local_i * _D + c * 16, 16)] = acc[c]

    for hh in range(2):
        hlo = hh * half
        b0 = hh * bh
        cps_h = _chunked_row_gather(h_hbm, node_v, hlo, hn_v, half, sem_h)
        issue(b0, 0)
        issue(b0 + 1, 1)

        def body(k, carry, b0=b0, hlo=hlo):
            for s in range(2):
                bidx = b0 + 2 * k + s
                drain(s)
                for r in range(_GB):
                    i = bidx * _GB + r
                    compute_row(i, i - hlo, s, r)

                @pl.when(bidx + 2 < b0 + bh)
                def _():
                    issue(bidx + 2, s)
            return carry

        lax.fori_loop(0, bh // 2, body, 0)

        for cp in cps_h:
            cp.wait()
        pltpu.sync_copy(hn_v, hn_out.at[pl.ds(base + hlo, half)])
        pltpu.sync_copy(agg_v,
                        agg_out.at[pl.ds((base + hlo) * _D, half * _D)])

    sid = lax.axis_index("s")
    cid = lax.axis_index("c")
    lo = sid * _NRANGE
    iota16 = lax.iota(jnp.int32, 16)
    last15 = iota16 == 15
    zero16 = jnp.zeros((16,), jnp.int32)
    nxt_idx = jnp.minimum(iota16 + 1, 15).reshape(16, 1)
    dnums = lax.GatherDimensionNumbers(
        offset_dims=(), collapsed_slice_dims=(0,), start_index_map=(0,))

    def ztab(k, carry):
        table_v[pl.ds(k * 16, 16)] = zero16
        return carry

    lax.fori_loop(0, _TBL // 16, ztab, 0)

    def scan_a(ch, carry):
        pltpu.sync_copy(node_hbm.at[pl.ds(ch * 2000, 2000)],
                        scan_v.at[pl.ds(0, 2000)])

        def grp_a(g, carry2, ch=ch):
            idx = scan_v[pl.ds(g * 16, 16)]
            key = idx * 16384 + (ch * 2000 + g * 16 + iota16)
            ks = lax.sort(key)
            idn = lax.shift_right_logical(ks, 14)
            jv = ks & 16383
            nxt = lax.gather(idn, nxt_idx, dnums, slice_sizes=(1,),
                             mode=lax.GatherScatterMode.PROMISE_IN_BOUNDS)
            m = ((idn != nxt) | last15) & (idn >= lo) & (idn < lo + _NRANGE)
            loc = jnp.clip(idn - lo, 0, _NRANGE - 1)
            plsc.store_scatter(table_v, [loc], jv, mask=m)
            return carry2

        lax.fori_loop(0, 125, grp_a, 0)
        return carry

    lax.fori_loop(0, 5, scan_a, 0)

    def scan_b(ch, carry):
        pltpu.sync_copy(node_hbm.at[pl.ds(ch * 2048, 2048)], scan_v)

        def grp_b(g, carry2, ch=ch):
            idx = scan_v[pl.ds(g * 16, 16)]
            loc = jnp.clip(idx - lo, 0, _NRANGE - 1)
            val = plsc.load_gather(table_v, [loc])
            inr = (idx >= lo) & (idx < lo + _NRANGE)
            contrib_v[pl.ds(ch * 2048 + g * 16, 16)] = \
                jnp.where(inr, val + 1, 0)
            return carry2

        lax.fori_loop(0, 2048 // 16, grp_b, 0)
        return carry

    lax.fori_loop(0, 5, scan_b, 0)
    pltpu.sync_copy(contrib_v, shared_v.at[sid])
    plsc.subcore_barrier()

    fpw = _HBP // _NS
    off = cid * _HBP + sid * fpw
    for s in range(_NS):
        pltpu.sync_copy(shared_v.at[s, pl.ds(off, fpw)], fin_acc.at[s])

    def fing(g, carry):
        tot = fin_acc[0, pl.ds(g * 16, 16)]
        for s in range(1, _NS):
            tot = tot + fin_acc[s, pl.ds(g * 16, 16)]
        fin_v[pl.ds(g * 16, 16)] = tot - 1
        return carry

    lax.fori_loop(0, fpw // 16, fing, 0)
    pltpu.sync_copy(fin_v, perm_out.at[pl.ds(off, fpw)])


def _sc_aggregate(hq, nb_p, w_p_flat, node_p, h):
    mesh = plsc.VectorSubcoreMesh(core_axis_name="c", subcore_axis_name="s")
    fn = functools.partial(
        pl.kernel,
        out_type=(
            jax.ShapeDtypeStruct((_BPAD * _D,), jnp.float32),
            jax.ShapeDtypeStruct((_BPAD,), jnp.int32),
            jax.ShapeDtypeStruct((_BPAD, _D), jnp.float32),
        ),
        mesh=mesh,
        scratch_types=[
            pltpu.VMEM((_BPW * _T,), jnp.int32),
            pltpu.VMEM((_BPW * _T,), jnp.float32),
            pltpu.VMEM((_BPW // 2 * _D,), jnp.float32),
            pltpu.VMEM((2, _BIDX, _D // 2), jnp.uint32),
            pltpu.VMEM((_BPW,), jnp.int32),
            pltpu.VMEM((_BPW // 2, _D), jnp.float32),
            pltpu.VMEM((_TBL,), jnp.int32),
            pltpu.VMEM((2048,), jnp.int32),
            pltpu.VMEM((_BPAD,), jnp.int32),
            pltpu.VMEM((_HBP // _NS,), jnp.int32),
            pltpu.VMEM((_NS, _HBP // _NS), jnp.int32),
            pltpu.VMEM_SHARED((_NS, _BPAD), jnp.int32),
            pltpu.SemaphoreType.DMA,
            pltpu.SemaphoreType.DMA,
            pltpu.SemaphoreType.DMA,
            pltpu.SemaphoreType.DMA,
            pltpu.SemaphoreType.DMA,
        ],
        compiler_params=pltpu.CompilerParams(needs_layout_passes=False,
                                             use_tc_tiling_on_sc=False),
    )(_sc_agg_body)
    return fn(hq, nb_p, w_p_flat, node_p, h)


def _sc_perm_body(src_hbm, perm_hbm, out_hbm, idx_v, rows_v, sem):
    base = _wid() * _BPW
    pltpu.sync_copy(perm_hbm.at[pl.ds(base, _BPW)], idx_v)
    for cp in _chunked_row_gather(src_hbm, idx_v, 0, rows_v, _BPW, sem):
        cp.wait()
    pltpu.sync_copy(rows_v, out_hbm.at[pl.ds(base, _BPW)])


def _sc_perm_gather(h_new, perm_p):
    mesh = plsc.VectorSubcoreMesh(core_axis_name="c", subcore_axis_name="s")
    fn = functools.partial(
        pl.kernel,
        out_type=jax.ShapeDtypeStruct((_BPAD, _D), jnp.float32),
        mesh=mesh,
        scratch_types=[
            pltpu.VMEM((_BPW,), jnp.int32),
            pltpu.VMEM((_BPW, _D), jnp.float32),
            pltpu.SemaphoreType.DMA,
        ],
    )(_sc_perm_body)
    return fn(h_new, perm_p)


def kernel(h, nodeset, nb_nodes, nb_weights, Q_w, Q_b, W_w, W_b):
    b, t = nb_nodes.shape
    pad = _BPAD - b
    spread = jnp.arange(pad, dtype=jnp.int32)
    node_p = jnp.concatenate([nodeset, spread])
    nb_p = jnp.concatenate(
        [nb_nodes.reshape(-1),
         jnp.arange(pad * t, dtype=jnp.int32) % _N])
    nb_p = jnp.where(nb_p < _N // 2, 2 * nb_p, 2 * nb_p - _N + 1)
    w_p_flat = jnp.concatenate(
        [nb_weights.reshape(-1), jnp.ones((pad * t,), jnp.float32)])

    hq = _hq_precompute(h, Q_w.T[:, _CHPERM], Q_b[_CHPERM].reshape(1, _D))
    agg_flat, perm_p, hn = _sc_aggregate(hq.reshape(_N, _D // 2), nb_p,
                                         w_p_flat, node_p, h)
    agg = agg_flat.reshape(_BPAD, _D)
    h_new = _out_layer(hn, agg, w_p_flat.reshape(_BPAD, _T),
                       W_w[:, :_D].T, W_w[:, _D:].T, W_b.reshape(1, _D))
    out = _sc_perm_gather(h_new, perm_p)
    return out[:b]


# speedup vs baseline: 1.3468x; 1.0079x over previous
import functools

import jax
import jax.numpy as jnp
import numpy as np
from jax import lax
from jax.experimental import pallas as pl
from jax.experimental.pallas import tpu as pltpu
from jax.experimental.pallas import tpu_sc as plsc

_N = 100000
_D = 128
_B = 10000
_T = 32

_NC = 2
_NS = 16
_NW = _NC * _NS
_BPAD = 10240
_BPW = _BPAD // _NW
_IDXCHUNK = 128


def _leaky(x):
    return jnp.where(x >= 0, x, 0.01 * x)


def _pack_u32(x):
    lo = lax.bitcast_convert_type(
        x[:, :_D // 2].astype(jnp.bfloat16), jnp.uint16).astype(jnp.uint32)
    hi = lax.bitcast_convert_type(
        x[:, _D // 2:].astype(jnp.bfloat16), jnp.uint16).astype(jnp.uint32)
    return lo | (hi << 16)


def _hq_body(ha_ref, hb_ref, qwt_ref, qb_ref, o_ref):
    xa = _leaky(jnp.dot(ha_ref[...], qwt_ref[...],
                        preferred_element_type=jnp.float32) + qb_ref[...])
    xb = _leaky(jnp.dot(hb_ref[...], qwt_ref[...],
                        preferred_element_type=jnp.float32) + qb_ref[...])
    o_ref[...] = jnp.concatenate([_pack_u32(xa), _pack_u32(xb)], axis=1)


def _hq_precompute(h, q_wt, q_b2):
    blk = 2000
    nb2 = _N // 2 // blk
    return pl.pallas_call(
        _hq_body,
        grid=(nb2,),
        in_specs=[
            pl.BlockSpec((blk, _D), lambda i: (i, 0)),
            pl.BlockSpec((blk, _D), lambda i: (i + nb2, 0)),
            pl.BlockSpec((_D, _D), lambda i: (0, 0)),
            pl.BlockSpec((1, _D), lambda i: (0, 0)),
        ],
        out_specs=pl.BlockSpec((blk, _D), lambda i: (i, 0)),
        out_shape=jax.ShapeDtypeStruct((_N // 2, _D), jnp.uint32),
    )(h, h, q_wt, q_b2)


def _out_body(hn_ref, agg_ref, w_ref, w1_ref, w2_ref, b_ref, o_ref):
    wsum = jnp.sum(w_ref[...], axis=1, keepdims=True)
    agg = agg_ref[...] / jnp.where(wsum == 0.0, 1.0, wsum)
    x = jnp.dot(hn_ref[...], w1_ref[...], preferred_element_type=jnp.float32)
    x = x + jnp.dot(agg, w2_ref[...], preferred_element_type=jnp.float32)
    x = _leaky(x + b_ref[...])
    nrm = jnp.sqrt(jnp.sum(x * x, axis=1, keepdims=True))
    o_ref[...] = x / jnp.where(nrm == 0.0, 1.0, nrm)


def _out_layer(hn, agg, w_p, w1t, w2t, w_b2):
    blk = 2048
    return pl.pallas_call(
        _out_body,
        grid=(_BPAD // blk,),
        in_specs=[
            pl.BlockSpec((blk, _D), lambda i: (i, 0)),
            pl.BlockSpec((blk, _D), lambda i: (i, 0)),
            pl.BlockSpec((blk, _T), lambda i: (i, 0)),
            pl.BlockSpec((_D, _D), lambda i: (0, 0)),
            pl.BlockSpec((_D, _D), lambda i: (0, 0)),
            pl.BlockSpec((1, _D), lambda i: (0, 0)),
        ],
        out_specs=pl.BlockSpec((blk, _D), lambda i: (i, 0)),
        out_shape=jax.ShapeDtypeStruct((_BPAD, _D), jnp.float32),
    )(hn, agg, w_p, w1t, w2t, w_b2)


def _wid():
    return lax.axis_index("s") * _NC + lax.axis_index("c")


def _chunked_row_gather(table_hbm, idx_v, idx_lo, dst_v, nrows, sem):
    copies = []
    for lo in range(0, nrows, _IDXCHUNK):
        n = min(_IDXCHUNK, nrows - lo)
        copies.append(pltpu.async_copy(
            table_hbm.at[idx_v.at[pl.ds(idx_lo + lo, n)]],
            dst_v.at[pl.ds(lo, n)], sem))
    return copies


_GB = 4
_BIDX = _GB * _T

_CHPERM = np.zeros(_D, dtype=np.int32)
for _g in range(4):
    for _i in range(16):
        _CHPERM[16 * _g + _i] = 32 * _g + _i
        _CHPERM[64 + 16 * _g + _i] = 32 * _g + 16 + _i


_NRANGE = _N // _NS
_TBL = 6256
_HBP = _BPAD // 2


def _sc_agg_body(hq_hbm, nb_hbm, w_hbm, node_hbm, h_hbm,
                 agg_out, perm_out, hn_out,
                 nb_v, w_v, agg_v, nbr_v, node_v, hn_v,
                 table_v, scan_v, contrib_v, fin_v, fin_acc, shared_v,
                 sem_nb, sem_w, sem_h, sem0, sem1):
    base = _wid() * _BPW
    half = _BPW // 2
    bh = half // _GB
    pltpu.sync_copy(node_hbm.at[pl.ds(base, _BPW)], node_v)
    cp_nb = pltpu.async_copy(nb_hbm.at[pl.ds(base * _T, _BPW * _T)], nb_v,
                             sem_nb)
    cp_w = pltpu.async_copy(w_hbm.at[pl.ds(base * _T, _BPW * _T)], w_v, sem_w)
    cp_nb.wait()
    cp_w.wait()

    sems = (sem0, sem1)

    def issue(bidx, s):
        pltpu.async_copy(
            hq_hbm.at[nb_v.at[pl.ds(bidx * _BIDX, _BIDX)]],
            nbr_v.at[s], sems[s])

    def drain(s):
        pltpu.make_async_copy(
            hq_hbm.at[nb_v.at[pl.ds(0, _BIDX)]], nbr_v.at[s], sems[s]).wait()

    def compute_row(i, local_i, buf, r):
        wr0 = w_v[pl.ds(i * _T, 16)]
        wr1 = w_v[pl.ds(i * _T + 16, 16)]
        acc = [jnp.zeros((16,), jnp.float32) for _ in range(8)]
        dnums = lax.GatherDimensionNumbers(
            offset_dims=(), collapsed_slice_dims=(0,), start_index_map=(0,))
        for t in range(_T):
            src = wr0 if t < 16 else wr1
            wt = lax.gather(src, jnp.full((16, 1), t % 16, jnp.int32),
                            dnums, slice_sizes=(1,),
                            mode=lax.GatherScatterMode.PROMISE_IN_BOUNDS)
            for g in range(4):
                u = nbr_v[buf, r * _T + t, pl.ds(g * 16, 16)]
                fe = plsc.bitcast(u << 16, jnp.float32)
                fo = plsc.bitcast(u, jnp.float32)
                acc[2 * g] = acc[2 * g] + wt * fe
                acc[2 * g + 1] = acc[2 * g + 1] + wt * fo
        for c in range(8):
            agg_v[pl.ds(local_i * _D + c * 16, 16)] = acc[c]

    for hh in range(2):
        hlo = hh * half
        b0 = hh * bh
        cps_h = _chunked_row_gather(h_hbm, node_v, hlo, hn_v, half, sem_h)
        issue(b0, 0)
        issue(b0 + 1, 1)

        def body(k, carry, b0=b0, hlo=hlo):
            for s in range(2):
                bidx = b0 + 2 * k + s
                drain(s)
                for r in range(_GB):
                    i = bidx * _GB + r
                    compute_row(i, i - hlo, s, r)

                @pl.when(bidx + 2 < b0 + bh)
                def _():
                    issue(bidx + 2, s)
            return carry

        lax.fori_loop(0, bh // 2, body, 0)

        for cp in cps_h:
            cp.wait()
        pltpu.sync_copy(hn_v, hn_out.at[pl.ds(base + hlo, half)])
        pltpu.sync_copy(agg_v,
                        agg_out.at[pl.ds((base + hlo) * _D, half * _D)])

    sid = lax.axis_index("s")
    cid = lax.axis_index("c")
    lo = sid * _NRANGE
    iota16 = lax.iota(jnp.int32, 16)
    last15 = iota16 == 15
    zero16 = jnp.zeros((16,), jnp.int32)
    nxt_idx = jnp.minimum(iota16 + 1, 15).reshape(16, 1)
    dnums = lax.GatherDimensionNumbers(
        offset_dims=(), collapsed_slice_dims=(0,), start_index_map=(0,))

    def ztab(k, carry):
        table_v[pl.ds(k * 16, 16)] = zero16
        return carry

    lax.fori_loop(0, _TBL // 16, ztab, 0)

    def scan_a(ch, carry):
        pltpu.sync_copy(node_hbm.at[pl.ds(ch * 2000, 2000)],
                        scan_v.at[pl.ds(0, 2000)])

        def grp_a(g5, carry2, ch=ch):
            for u in range(5):
                g = g5 * 5 + u
                idx = scan_v[pl.ds(g * 16, 16)]
                key = idx * 16384 + (ch * 2000 + g * 16 + iota16)
                ks = lax.sort(key)
                idn = lax.shift_right_logical(ks, 14)
                jv = ks & 16383
                nxt = lax.gather(idn, nxt_idx, dnums, slice_sizes=(1,),
                                 mode=lax.GatherScatterMode.PROMISE_IN_BOUNDS)
                m = (((idn != nxt) | last15) & (idn >= lo)
                     & (idn < lo + _NRANGE))
                loc = jnp.clip(idn - lo, 0, _NRANGE - 1)
                plsc.store_scatter(table_v, [loc], jv, mask=m)
            return carry2

        lax.fori_loop(0, 25, grp_a, 0)
        return carry

    lax.fori_loop(0, 5, scan_a, 0)

    def scan_b(ch, carry):
        pltpu.sync_copy(node_hbm.at[pl.ds(ch * 2048, 2048)], scan_v)

        def grp_b(g4, carry2, ch=ch):
            for u in range(4):
                g = g4 * 4 + u
                idx = scan_v[pl.ds(g * 16, 16)]
                loc = jnp.clip(idx - lo, 0, _NRANGE - 1)
                val = plsc.load_gather(table_v, [loc])
                inr = (idx >= lo) & (idx < lo + _NRANGE)
                contrib_v[pl.ds(ch * 2048 + g * 16, 16)] = \
                    jnp.where(inr, val + 1, 0)
            return carry2

        lax.fori_loop(0, 2048 // 16 // 4, grp_b, 0)
        return carry

    lax.fori_loop(0, 5, scan_b, 0)
    pltpu.sync_copy(contrib_v, shared_v.at[sid])
    plsc.subcore_barrier()

    fpw = _HBP // _NS
    off = cid * _HBP + sid * fpw
    for s in range(_NS):
        pltpu.sync_copy(shared_v.at[s, pl.ds(off, fpw)], fin_acc.at[s])

    def fing(g, carry):
        tot = fin_acc[0, pl.ds(g * 16, 16)]
        for s in range(1, _NS):
            tot = tot + fin_acc[s, pl.ds(g * 16, 16)]
        fin_v[pl.ds(g * 16, 16)] = tot - 1
        return carry

    lax.fori_loop(0, fpw // 16, fing, 0)
    pltpu.sync_copy(fin_v, perm_out.at[pl.ds(off, fpw)])


def _sc_aggregate(hq, nb_p, w_p_flat, node_p, h):
    mesh = plsc.VectorSubcoreMesh(core_axis_name="c", subcore_axis_name="s")
    fn = functools.partial(
        pl.kernel,
        out_type=(
            jax.ShapeDtypeStruct((_BPAD * _D,), jnp.float32),
            jax.ShapeDtypeStruct((_BPAD,), jnp.int32),
            jax.ShapeDtypeStruct((_BPAD, _D), jnp.float32),
        ),
        mesh=mesh,
        scratch_types=[
            pltpu.VMEM((_BPW * _T,), jnp.int32),
            pltpu.VMEM((_BPW * _T,), jnp.float32),
            pltpu.VMEM((_BPW // 2 * _D,), jnp.float32),
            pltpu.VMEM((2, _BIDX, _D // 2), jnp.uint32),
            pltpu.VMEM((_BPW,), jnp.int32),
            pltpu.VMEM((_BPW // 2, _D), jnp.float32),
            pltpu.VMEM((_TBL,), jnp.int32),
            pltpu.VMEM((2048,), jnp.int32),
            pltpu.VMEM((_BPAD,), jnp.int32),
            pltpu.VMEM((_HBP // _NS,), jnp.int32),
            pltpu.VMEM((_NS, _HBP // _NS), jnp.int32),
            pltpu.VMEM_SHARED((_NS, _BPAD), jnp.int32),
            pltpu.SemaphoreType.DMA,
            pltpu.SemaphoreType.DMA,
            pltpu.SemaphoreType.DMA,
            pltpu.SemaphoreType.DMA,
            pltpu.SemaphoreType.DMA,
        ],
        compiler_params=pltpu.CompilerParams(needs_layout_passes=False,
                                             use_tc_tiling_on_sc=False),
    )(_sc_agg_body)
    return fn(hq, nb_p, w_p_flat, node_p, h)


def _sc_perm_body(src_hbm, perm_hbm, out_hbm, idx_v, rows_v, sem):
    base = _wid() * _BPW
    pltpu.sync_copy(perm_hbm.at[pl.ds(base, _BPW)], idx_v)
    for cp in _chunked_row_gather(src_hbm, idx_v, 0, rows_v, _BPW, sem):
        cp.wait()
    pltpu.sync_copy(rows_v, out_hbm.at[pl.ds(base, _BPW)])


def _sc_perm_gather(h_new, perm_p):
    mesh = plsc.VectorSubcoreMesh(core_axis_name="c", subcore_axis_name="s")
    fn = functools.partial(
        pl.kernel,
        out_type=jax.ShapeDtypeStruct((_BPAD, _D), jnp.float32),
        mesh=mesh,
        scratch_types=[
            pltpu.VMEM((_BPW,), jnp.int32),
            pltpu.VMEM((_BPW, _D), jnp.float32),
            pltpu.SemaphoreType.DMA,
        ],
    )(_sc_perm_body)
    return fn(h_new, perm_p)


def kernel(h, nodeset, nb_nodes, nb_weights, Q_w, Q_b, W_w, W_b):
    b, t = nb_nodes.shape
    pad = _BPAD - b
    spread = jnp.arange(pad, dtype=jnp.int32)
    node_p = jnp.concatenate([nodeset, spread])
    nb_p = jnp.concatenate(
        [nb_nodes.reshape(-1),
         jnp.arange(pad * t, dtype=jnp.int32) % _N])
    nb_p = jnp.where(nb_p < _N // 2, 2 * nb_p, 2 * nb_p - _N + 1)
    w_p_flat = jnp.concatenate(
        [nb_weights.reshape(-1), jnp.ones((pad * t,), jnp.float32)])

    hq = _hq_precompute(h, Q_w.T[:, _CHPERM], Q_b[_CHPERM].reshape(1, _D))
    agg_flat, perm_p, hn = _sc_aggregate(hq.reshape(_N, _D // 2), nb_p,
                                         w_p_flat, node_p, h)
    agg = agg_flat.reshape(_BPAD, _D)
    h_new = _out_layer(hn, agg, w_p_flat.reshape(_BPAD, _T),
                       W_w[:, :_D].T, W_w[:, _D:].T, W_b.reshape(1, _D))
    out = _sc_perm_gather(h_new, perm_p)
    return out[:b]


# speedup vs baseline: 1.3522x; 1.0041x over previous
import functools

import jax
import jax.numpy as jnp
import numpy as np
from jax import lax
from jax.experimental import pallas as pl
from jax.experimental.pallas import tpu as pltpu
from jax.experimental.pallas import tpu_sc as plsc

_N = 100000
_D = 128
_B = 10000
_T = 32

_NC = 2
_NS = 16
_NW = _NC * _NS
_BPAD = 10240
_BPW = _BPAD // _NW
_IDXCHUNK = 128


def _leaky(x):
    return jnp.where(x >= 0, x, 0.01 * x)


def _pack_u32(x):
    lo = lax.bitcast_convert_type(
        x[:, :_D // 2].astype(jnp.bfloat16), jnp.uint16).astype(jnp.uint32)
    hi = lax.bitcast_convert_type(
        x[:, _D // 2:].astype(jnp.bfloat16), jnp.uint16).astype(jnp.uint32)
    return lo | (hi << 16)


def _hq_body(ha_ref, hb_ref, qwt_ref, qb_ref, o_ref):
    xa = _leaky(jnp.dot(ha_ref[...], qwt_ref[...],
                        preferred_element_type=jnp.float32) + qb_ref[...])
    xb = _leaky(jnp.dot(hb_ref[...], qwt_ref[...],
                        preferred_element_type=jnp.float32) + qb_ref[...])
    o_ref[...] = jnp.concatenate([_pack_u32(xa), _pack_u32(xb)], axis=1)


def _hq_precompute(h, q_wt, q_b2):
    blk = 2000
    nb2 = _N // 2 // blk
    return pl.pallas_call(
        _hq_body,
        grid=(nb2,),
        in_specs=[
            pl.BlockSpec((blk, _D), lambda i: (i, 0)),
            pl.BlockSpec((blk, _D), lambda i: (i + nb2, 0)),
            pl.BlockSpec((_D, _D), lambda i: (0, 0)),
            pl.BlockSpec((1, _D), lambda i: (0, 0)),
        ],
        out_specs=pl.BlockSpec((blk, _D), lambda i: (i, 0)),
        out_shape=jax.ShapeDtypeStruct((_N // 2, _D), jnp.uint32),
    )(h, h, q_wt, q_b2)


def _out_body(hn_ref, agg_ref, w_ref, w1_ref, w2_ref, b_ref, o_ref):
    wsum = jnp.sum(w_ref[...], axis=1, keepdims=True)
    agg = agg_ref[...] / jnp.where(wsum == 0.0, 1.0, wsum)
    x = jnp.dot(hn_ref[...], w1_ref[...], preferred_element_type=jnp.float32)
    x = x + jnp.dot(agg, w2_ref[...], preferred_element_type=jnp.float32)
    x = _leaky(x + b_ref[...])
    nrm = jnp.sqrt(jnp.sum(x * x, axis=1, keepdims=True))
    o_ref[...] = x / jnp.where(nrm == 0.0, 1.0, nrm)


def _out_layer(hn, agg, w_p, w1t, w2t, w_b2):
    blk = 2048
    return pl.pallas_call(
        _out_body,
        grid=(_BPAD // blk,),
        in_specs=[
            pl.BlockSpec((blk, _D), lambda i: (i, 0)),
            pl.BlockSpec((blk, _D), lambda i: (i, 0)),
            pl.BlockSpec((blk, _T), lambda i: (i, 0)),
            pl.BlockSpec((_D, _D), lambda i: (0, 0)),
            pl.BlockSpec((_D, _D), lambda i: (0, 0)),
            pl.BlockSpec((1, _D), lambda i: (0, 0)),
        ],
        out_specs=pl.BlockSpec((blk, _D), lambda i: (i, 0)),
        out_shape=jax.ShapeDtypeStruct((_BPAD, _D), jnp.float32),
    )(hn, agg, w_p, w1t, w2t, w_b2)


def _wid():
    return lax.axis_index("s") * _NC + lax.axis_index("c")


def _chunked_row_gather(table_hbm, idx_v, idx_lo, dst_v, nrows, sem):
    copies = []
    for lo in range(0, nrows, _IDXCHUNK):
        n = min(_IDXCHUNK, nrows - lo)
        copies.append(pltpu.async_copy(
            table_hbm.at[idx_v.at[pl.ds(idx_lo + lo, n)]],
            dst_v.at[pl.ds(lo, n)], sem))
    return copies


_GB = 4
_BIDX = _GB * _T

_CHPERM = np.zeros(_D, dtype=np.int32)
for _g in range(4):
    for _i in range(16):
        _CHPERM[16 * _g + _i] = 32 * _g + _i
        _CHPERM[64 + 16 * _g + _i] = 32 * _g + 16 + _i


_NRANGE = _N // _NS
_TBL = 6256
_HBP = _BPAD // 2


def _sc_agg_body(hq_hbm, nb_hbm, w_hbm, node_hbm, h_hbm,
                 agg_out, hn_out,
                 nb_v, w_v, agg_v, nbr_v, node_v, hn_v,
                 sem_nb, sem_w, sem_h, sem0, sem1):
    base = _wid() * _BPW
    half = _BPW // 2
    bh = half // _GB
    pltpu.sync_copy(node_hbm.at[pl.ds(base, _BPW)], node_v)
    cp_nb = pltpu.async_copy(nb_hbm.at[pl.ds(base * _T, _BPW * _T)], nb_v,
                             sem_nb)
    cp_w = pltpu.async_copy(w_hbm.at[pl.ds(base * _T, _BPW * _T)], w_v, sem_w)
    cp_nb.wait()
    cp_w.wait()

    sems = (sem0, sem1)

    def issue(bidx, s):
        pltpu.async_copy(
            hq_hbm.at[nb_v.at[pl.ds(bidx * _BIDX, _BIDX)]],
            nbr_v.at[s], sems[s])

    def drain(s):
        pltpu.make_async_copy(
            hq_hbm.at[nb_v.at[pl.ds(0, _BIDX)]], nbr_v.at[s], sems[s]).wait()

    def compute_row(i, local_i, buf, r):
        wr0 = w_v[pl.ds(i * _T, 16)]
        wr1 = w_v[pl.ds(i * _T + 16, 16)]
        acc = [jnp.zeros((16,), jnp.float32) for _ in range(8)]
        dnums = lax.GatherDimensionNumbers(
            offset_dims=(), collapsed_slice_dims=(0,), start_index_map=(0,))
        for t in range(_T):
            src = wr0 if t < 16 else wr1
            wt = lax.gather(src, jnp.full((16, 1), t % 16, jnp.int32),
                            dnums, slice_sizes=(1,),
                            mode=lax.GatherScatterMode.PROMISE_IN_BOUNDS)
            for g in range(4):
                u = nbr_v[buf, r * _T + t, pl.ds(g * 16, 16)]
                fe = plsc.bitcast(u << 16, jnp.float32)
                fo = plsc.bitcast(u, jnp.float32)
                acc[2 * g] = acc[2 * g] + wt * fe
                acc[2 * g + 1] = acc[2 * g + 1] + wt * fo
        for c in range(8):
            agg_v[pl.ds(local_i * _D + c * 16, 16)] = acc[c]

    for hh in range(2):
        hlo = hh * half
        b0 = hh * bh
        cps_h = _chunked_row_gather(h_hbm, node_v, hlo, hn_v, half, sem_h)
        issue(b0, 0)
        issue(b0 + 1, 1)

        def body(k, carry, b0=b0, hlo=hlo):
            for s in range(2):
                bidx = b0 + 2 * k + s
                drain(s)
                for r in range(_GB):
                    i = bidx * _GB + r
                    compute_row(i, i - hlo, s, r)

                @pl.when(bidx + 2 < b0 + bh)
                def _():
                    issue(bidx + 2, s)
            return carry

        lax.fori_loop(0, bh // 2, body, 0)

        for cp in cps_h:
            cp.wait()
        pltpu.sync_copy(hn_v, hn_out.at[pl.ds(base + hlo, half)])
        pltpu.sync_copy(agg_v,
                        agg_out.at[pl.ds((base + hlo) * _D, half * _D)])


def _sc_winner_body(node_hbm, perm_out,
                    table_v, scan_v, contrib_v, fin_v, fin_acc, shared_v):
    sid = lax.axis_index("s")
    cid = lax.axis_index("c")
    lo = sid * _NRANGE
    iota16 = lax.iota(jnp.int32, 16)
    last15 = iota16 == 15
    zero16 = jnp.zeros((16,), jnp.int32)
    nxt_idx = jnp.minimum(iota16 + 1, 15).reshape(16, 1)
    dnums = lax.GatherDimensionNumbers(
        offset_dims=(), collapsed_slice_dims=(0,), start_index_map=(0,))

    def ztab(k, carry):
        table_v[pl.ds(k * 16, 16)] = zero16
        return carry

    lax.fori_loop(0, _TBL // 16, ztab, 0)

    def scan_a(ch, carry):
        pltpu.sync_copy(node_hbm.at[pl.ds(ch * 2000, 2000)],
                        scan_v.at[pl.ds(0, 2000)])

        def grp_a(g5, carry2, ch=ch):
            for u in range(5):
                g = g5 * 5 + u
                idx = scan_v[pl.ds(g * 16, 16)]
                key = idx * 16384 + (ch * 2000 + g * 16 + iota16)
                ks = lax.sort(key)
                idn = lax.shift_right_logical(ks, 14)
                jv = ks & 16383
                nxt = lax.gather(idn, nxt_idx, dnums, slice_sizes=(1,),
                                 mode=lax.GatherScatterMode.PROMISE_IN_BOUNDS)
                m = (((idn != nxt) | last15) & (idn >= lo)
                     & (idn < lo + _NRANGE))
                loc = jnp.clip(idn - lo, 0, _NRANGE - 1)
                plsc.store_scatter(table_v, [loc], jv, mask=m)
            return carry2

        lax.fori_loop(0, 25, grp_a, 0)
        return carry

    lax.fori_loop(0, 5, scan_a, 0)

    def scan_b(ch, carry):
        pltpu.sync_copy(node_hbm.at[pl.ds(ch * 2048, 2048)], scan_v)

        def grp_b(g4, carry2, ch=ch):
            for u in range(4):
                g = g4 * 4 + u
                idx = scan_v[pl.ds(g * 16, 16)]
                loc = jnp.clip(idx - lo, 0, _NRANGE - 1)
                val = plsc.load_gather(table_v, [loc])
                inr = (idx >= lo) & (idx < lo + _NRANGE)
                contrib_v[pl.ds(ch * 2048 + g * 16, 16)] = \
                    jnp.where(inr, val + 1, 0)
            return carry2

        lax.fori_loop(0, 2048 // 16 // 4, grp_b, 0)
        return carry

    lax.fori_loop(0, 5, scan_b, 0)
    pltpu.sync_copy(contrib_v, shared_v.at[sid])
    plsc.subcore_barrier()

    fpw = _HBP // _NS
    off = cid * _HBP + sid * fpw
    for s in range(_NS):
        pltpu.sync_copy(shared_v.at[s, pl.ds(off, fpw)], fin_acc.at[s])

    def fing(g, carry):
        tot = fin_acc[0, pl.ds(g * 16, 16)]
        for s in range(1, _NS):
            tot = tot + fin_acc[s, pl.ds(g * 16, 16)]
        fin_v[pl.ds(g * 16, 16)] = tot - 1
        return carry

    lax.fori_loop(0, fpw // 16, fing, 0)
    pltpu.sync_copy(fin_v, perm_out.at[pl.ds(off, fpw)])


def _sc_aggregate(hq, nb_p, w_p_flat, node_p, h):
    mesh = plsc.VectorSubcoreMesh(core_axis_name="c", subcore_axis_name="s")
    fn = functools.partial(
        pl.kernel,
        out_type=(
            jax.ShapeDtypeStruct((_BPAD * _D,), jnp.float32),
            jax.ShapeDtypeStruct((_BPAD, _D), jnp.float32),
        ),
        mesh=mesh,
        scratch_types=[
            pltpu.VMEM((_BPW * _T,), jnp.int32),
            pltpu.VMEM((_BPW * _T,), jnp.float32),
            pltpu.VMEM((_BPW // 2 * _D,), jnp.float32),
            pltpu.VMEM((2, _BIDX, _D // 2), jnp.uint32),
            pltpu.VMEM((_BPW,), jnp.int32),
            pltpu.VMEM((_BPW // 2, _D), jnp.float32),
            pltpu.SemaphoreType.DMA,
            pltpu.SemaphoreType.DMA,
            pltpu.SemaphoreType.DMA,
            pltpu.SemaphoreType.DMA,
            pltpu.SemaphoreType.DMA,
        ],
        compiler_params=pltpu.CompilerParams(needs_layout_passes=False,
                                             use_tc_tiling_on_sc=False),
    )(_sc_agg_body)
    return fn(hq, nb_p, w_p_flat, node_p, h)


def _sc_winner(node_p):
    mesh = plsc.VectorSubcoreMesh(core_axis_name="c", subcore_axis_name="s")
    fn = functools.partial(
        pl.kernel,
        out_type=jax.ShapeDtypeStruct((_BPAD,), jnp.int32),
        mesh=mesh,
        scratch_types=[
            pltpu.VMEM((_TBL,), jnp.int32),
            pltpu.VMEM((2048,), jnp.int32),
            pltpu.VMEM((_BPAD,), jnp.int32),
            pltpu.VMEM((_HBP // _NS,), jnp.int32),
            pltpu.VMEM((_NS, _HBP // _NS), jnp.int32),
            pltpu.VMEM_SHARED((_NS, _BPAD), jnp.int32),
        ],
        compiler_params=pltpu.CompilerParams(needs_layout_passes=False,
                                             use_tc_tiling_on_sc=False),
    )(_sc_winner_body)
    return fn(node_p)


def _sc_perm_body(src_hbm, perm_hbm, out_hbm, idx_v, rows_v, sem):
    base = _wid() * _BPW
    pltpu.sync_copy(perm_hbm.at[pl.ds(base, _BPW)], idx_v)
    for cp in _chunked_row_gather(src_hbm, idx_v, 0, rows_v, _BPW, sem):
        cp.wait()
    pltpu.sync_copy(rows_v, out_hbm.at[pl.ds(base, _BPW)])


def _sc_perm_gather(h_new, perm_p):
    mesh = plsc.VectorSubcoreMesh(core_axis_name="c", subcore_axis_name="s")
    fn = functools.partial(
        pl.kernel,
        out_type=jax.ShapeDtypeStruct((_BPAD, _D), jnp.float32),
        mesh=mesh,
        scratch_types=[
            pltpu.VMEM((_BPW,), jnp.int32),
            pltpu.VMEM((_BPW, _D), jnp.float32),
            pltpu.SemaphoreType.DMA,
        ],
    )(_sc_perm_body)
    return fn(h_new, perm_p)


def kernel(h, nodeset, nb_nodes, nb_weights, Q_w, Q_b, W_w, W_b):
    b, t = nb_nodes.shape
    pad = _BPAD - b
    spread = jnp.arange(pad, dtype=jnp.int32)
    node_p = jnp.concatenate([nodeset, spread])
    nb_p = jnp.concatenate(
        [nb_nodes.reshape(-1),
         jnp.arange(pad * t, dtype=jnp.int32) % _N])
    nb_p = jnp.where(nb_p < _N // 2, 2 * nb_p, 2 * nb_p - _N + 1)
    w_p_flat = jnp.concatenate(
        [nb_weights.reshape(-1), jnp.ones((pad * t,), jnp.float32)])

    perm_p = _sc_winner(node_p)
    hq = _hq_precompute(h, Q_w.T[:, _CHPERM], Q_b[_CHPERM].reshape(1, _D))
    agg_flat, hn = _sc_aggregate(hq.reshape(_N, _D // 2), nb_p,
                                 w_p_flat, node_p, h)
    agg = agg_flat.reshape(_BPAD, _D)
    h_new = _out_layer(hn, agg, w_p_flat.reshape(_BPAD, _T),
                       W_w[:, :_D].T, W_w[:, _D:].T, W_b.reshape(1, _D))
    out = _sc_perm_gather(h_new, perm_p)
    return out[:b]
